# baseline (speedup 1.0000x reference)
import functools

import jax
import jax.numpy as jnp
from jax.experimental import pallas as pl
from jax.experimental.pallas import tpu as pltpu
from jax.experimental.pallas import tpu_sc as plsc

HEADS = 8
DIM = 64
K = 1024
TB = 1024
NP = HEADS // 2
AUG = 144


def _argmin_body(x_ref, e_ref, idx_ref, ea_ref):
    xb = x_ref[...]
    em = e_ref[...]

    @pl.when((pl.program_id(0) == 0) & (pl.program_id(1) == 0))
    def _():
        em16 = (em * -2.0).astype(jnp.bfloat16)
        z = jnp.zeros((K, DIM), jnp.bfloat16)
        y2 = jnp.sum(em * em, axis=1)
        h1 = y2.astype(jnp.bfloat16)
        r1 = y2 - h1.astype(jnp.float32)
        h2 = r1.astype(jnp.bfloat16)
        h3 = (r1 - h2.astype(jnp.float32)).astype(jnp.bfloat16)
        ztail = jnp.zeros((K, AUG - 2 * DIM - 3), jnp.bfloat16)
        top = jnp.concatenate(
            [em16, z, h1[:, None], h2[:, None], h3[:, None], ztail], axis=1)
        bot = jnp.concatenate(
            [z, em16, h1[:, None], h2[:, None], h3[:, None], ztail], axis=1)
        ea_ref[...] = jnp.concatenate([top, bot], axis=0)

    xa = jnp.concatenate(
        [xb.astype(jnp.bfloat16),
         jnp.ones((TB, 3), jnp.bfloat16),
         jnp.zeros((TB, AUG - 2 * DIM - 3), jnp.bfloat16)], axis=1)
    s = jax.lax.dot_general(
        ea_ref[...], xa,
        (((1,), (1,)), ((), ())),
        preferred_element_type=jnp.float32)
    ie = jnp.argmin(s[:K], axis=0).astype(jnp.int32)
    io = jnp.argmin(s[K:], axis=0).astype(jnp.int32)
    p = pl.program_id(1)
    lane = jax.lax.broadcasted_iota(jnp.int32, (TB, HEADS), 1)
    vals = jnp.where(lane == 2 * p, ie[:, None],
                     jnp.where(lane == 2 * p + 1, io[:, None], idx_ref[...]))
    idx_ref[...] = vals


def _tc_argmin(x2d, em, interpret=False):
    nt = x2d.shape[0]
    nb = nt // TB
    idx2 = pl.pallas_call(
        _argmin_body,
        grid=(nb, NP),
        in_specs=[
            pl.BlockSpec((TB, 2 * DIM), lambda t, p: (t, p)),
            pl.BlockSpec((K, DIM), lambda t, p: (0, 0)),
        ],
        out_specs=pl.BlockSpec((TB, HEADS), lambda t, p: (t, 0)),
        out_shape=jax.ShapeDtypeStruct((nt, HEADS), jnp.int32),
        scratch_shapes=[pltpu.VMEM((2 * K, AUG), jnp.bfloat16)],
        interpret=interpret,
    )(x2d, em)
    return idx2.reshape(nt * HEADS)


def _sc_gather(epad, ind2):
    m = ind2.shape[1]
    nt = m // HEADS
    w = 256
    wt = w // HEADS

    @functools.partial(
        pl.kernel,
        out_type=jax.ShapeDtypeStruct((nt, HEADS * DIM), jnp.float32),
        mesh=plsc.VectorSubcoreMesh(core_axis_name="c", subcore_axis_name="s"),
        scratch_types=[pltpu.VMEM_SHARED((K, 2 * DIM), jnp.float32),
                       pltpu.VMEM((w, 2 * DIM), jnp.float32)],
    )
    def gather_kernel(e_hbm, i_hbm, o_hbm, tbl_ref, g_ref):
        @pl.when(jax.lax.axis_index("s") == 0)
        def _():
            pltpu.sync_copy(e_hbm, tbl_ref)
        plsc.subcore_barrier()

        def body(i_vmem, o_vmem):
            pltpu.sync_copy(tbl_ref.at[i_vmem.at[0]], g_ref)
            o_vmem[...] = g_ref[:, 0:DIM].reshape(wt, HEADS * DIM)

        pltpu.emit_pipeline(
            body,
            grid=(m // w,),
            in_specs=[pl.BlockSpec((1, w), index_map=lambda i: (0, i))],
            out_specs=[pl.BlockSpec((wt, HEADS * DIM),
                                    index_map=lambda i: (i, 0))],
            core_axis_name=("c", "s"),
            dimension_semantics=(pltpu.PARALLEL,),
        )(i_hbm, o_hbm)

    return gather_kernel(epad, ind2)


NCHUNK = 4


def kernel(x, embed):
    b, n, _ = x.shape
    nt = b * n
    em = embed[0]
    x2d = x.reshape(nt, HEADS * DIM)
    epad = jnp.pad(em, ((0, 0), (0, DIM)))
    ct = nt // NCHUNK
    inds, qs = [], []
    for c in range(NCHUNK):
        ind_c = _tc_argmin(x2d[c * ct:(c + 1) * ct], em)
        inds.append(ind_c)
        qs.append(_sc_gather(epad, ind_c.reshape(1, ct * HEADS)))
    q = jnp.concatenate(qs, axis=0)
    ind = jnp.concatenate(inds, axis=0)
    return q.reshape(b, n, HEADS * DIM), ind.reshape(b, n, HEADS)


# speedup vs baseline: 1.2243x; 1.2243x over previous
import functools

import jax
import jax.numpy as jnp
from jax.experimental import pallas as pl
from jax.experimental.pallas import tpu as pltpu
from jax.experimental.pallas import tpu_sc as plsc

HEADS = 8
DIM = 64
K = 1024
TB = 2048
NP = HEADS // 2
AUG = 144


def _argmin_body(x_ref, e_ref, idx_ref, ea_ref):
    xb = x_ref[...]
    em = e_ref[...]

    @pl.when((pl.program_id(0) == 0) & (pl.program_id(1) == 0))
    def _():
        em16 = (em * -2.0).astype(jnp.bfloat16)
        z = jnp.zeros((K, DIM), jnp.bfloat16)
        y2 = jnp.sum(em * em, axis=1)
        h1 = y2.astype(jnp.bfloat16)
        r1 = y2 - h1.astype(jnp.float32)
        h2 = r1.astype(jnp.bfloat16)
        h3 = (r1 - h2.astype(jnp.float32)).astype(jnp.bfloat16)
        ztail = jnp.zeros((K, AUG - 2 * DIM - 3), jnp.bfloat16)
        top = jnp.concatenate(
            [em16, z, h1[:, None], h2[:, None], h3[:, None], ztail], axis=1)
        bot = jnp.concatenate(
            [z, em16, h1[:, None], h2[:, None], h3[:, None], ztail], axis=1)
        ea_ref[...] = jnp.concatenate([top, bot], axis=0)

    xa = jnp.concatenate(
        [xb.astype(jnp.bfloat16),
         jnp.ones((TB, 3), jnp.bfloat16),
         jnp.zeros((TB, AUG - 2 * DIM - 3), jnp.bfloat16)], axis=1)
    s = jax.lax.dot_general(
        ea_ref[...], xa,
        (((1,), (1,)), ((), ())),
        preferred_element_type=jnp.float32)
    ie = jnp.argmin(s[:K], axis=0).astype(jnp.int32)
    io = jnp.argmin(s[K:], axis=0).astype(jnp.int32)
    p = pl.program_id(1)
    lane = jax.lax.broadcasted_iota(jnp.int32, (TB, HEADS), 1)
    vals = jnp.where(lane == 2 * p, ie[:, None],
                     jnp.where(lane == 2 * p + 1, io[:, None], idx_ref[...]))
    idx_ref[...] = vals


def _tc_argmin(x2d, em, interpret=False):
    nt = x2d.shape[0]
    nb = nt // TB
    idx2 = pl.pallas_call(
        _argmin_body,
        grid=(nb, NP),
        in_specs=[
            pl.BlockSpec((TB, 2 * DIM), lambda t, p: (t, p)),
            pl.BlockSpec((K, DIM), lambda t, p: (0, 0)),
        ],
        out_specs=pl.BlockSpec((TB, HEADS), lambda t, p: (t, 0)),
        out_shape=jax.ShapeDtypeStruct((nt, HEADS), jnp.int32),
        scratch_shapes=[pltpu.VMEM((2 * K, AUG), jnp.bfloat16)],
        interpret=interpret,
    )(x2d, em)
    return idx2.reshape(nt * HEADS)


def _sc_gather(epad, ind2):
    m = ind2.shape[1]
    nt = m // HEADS
    w = 256
    wt = w // HEADS

    @functools.partial(
        pl.kernel,
        out_type=jax.ShapeDtypeStruct((nt, HEADS * DIM), jnp.float32),
        mesh=plsc.VectorSubcoreMesh(core_axis_name="c", subcore_axis_name="s"),
        scratch_types=[pltpu.VMEM_SHARED((K, 2 * DIM), jnp.float32),
                       pltpu.VMEM((w, 2 * DIM), jnp.float32)],
    )
    def gather_kernel(e_hbm, i_hbm, o_hbm, tbl_ref, g_ref):
        @pl.when(jax.lax.axis_index("s") == 0)
        def _():
            pltpu.sync_copy(e_hbm, tbl_ref)
        plsc.subcore_barrier()

        def body(i_vmem, o_vmem):
            pltpu.sync_copy(tbl_ref.at[i_vmem.at[0]], g_ref)
            o_vmem[...] = g_ref[:, 0:DIM].reshape(wt, HEADS * DIM)

        pltpu.emit_pipeline(
            body,
            grid=(m // w,),
            in_specs=[pl.BlockSpec((1, w), index_map=lambda i: (0, i))],
            out_specs=[pl.BlockSpec((wt, HEADS * DIM),
                                    index_map=lambda i: (i, 0))],
            core_axis_name=("c", "s"),
            dimension_semantics=(pltpu.PARALLEL,),
        )(i_hbm, o_hbm)

    return gather_kernel(epad, ind2)


def kernel(x, embed):
    b, n, _ = x.shape
    nt = b * n
    em = embed[0]
    x2d = x.reshape(nt, HEADS * DIM)
    ind = _tc_argmin(x2d, em)
    epad = jnp.pad(em, ((0, 0), (0, DIM)))
    q = _sc_gather(epad, ind.reshape(1, nt * HEADS))
    return q.reshape(b, n, HEADS * DIM), ind.reshape(b, n, HEADS)
